# baseline (speedup 1.0000x reference)
import functools

import jax
import jax.numpy as jnp
from jax import lax
from jax.experimental import pallas as pl
from jax.experimental.pallas import tpu as pltpu
from jax.experimental.pallas import tpu_sc as plsc

NUM_BINS = 256
IMG_B, IMG_C, IMG_H, IMG_W = 16, 3, 512, 512
N_CH = IMG_B * IMG_C
LANES = 16
UNITS_PER_TILE = 3
HALF_ROWS = IMG_H // 2
ROWS = 32
CHUNK = ROWS * IMG_W
N_CHUNKS = HALF_ROWS // ROWS
STEPS = 2 * N_CHUNKS
VREGS = CHUNK // LANES
HSIZE = NUM_BINS * LANES
NBUF = 4


def _hist_body(pred_hbm, target_hbm, out_hbm, buf0, buf1, buf2, buf3, hist,
               outbuf, sem0, sem1, sem2, sem3):
    cid = lax.axis_index("c")
    sid = lax.axis_index("s")
    wid = sid * 2 + cid
    lane = lax.iota(jnp.int32, LANES)
    lane_t = lane | (1 << 12)
    ones = jnp.ones((LANES,), jnp.float32)
    zeros = jnp.zeros((LANES,), jnp.float32)
    bufs = (buf0, buf1, buf2, buf3)
    sems = (sem0, sem1, sem2, sem3)
    u0 = wid * UNITS_PER_TILE

    def start_copy(q, row0):
        src = pred_hbm if q % 2 == 0 else target_hbm
        pltpu.async_copy(src.at[pl.ds(row0, ROWS), :], bufs[q], sems[q])

    def wait_copy(q):
        pltpu.make_async_copy(
            pred_hbm.at[pl.ds(0, ROWS), :], bufs[q], sems[q]).wait()

    def row_of(t_unit, t_step):
        return (u0 + t_unit) * HALF_ROWS + (t_step // 2) * ROWS

    for q in range(NBUF):
        start_copy(q, row_of(0, q))

    for j in range(UNITS_PER_TILE):

        @plsc.parallel_loop(0, 2 * NUM_BINS, 1, unroll=8)
        def _(k):
            hist[pl.ds(k * LANES, LANES)] = zeros

        def quad_body(s4, carry):
            for q in range(NBUF):
                wait_copy(q)
                buf = bufs[q]
                region = 0 if q % 2 == 0 else HSIZE

                @plsc.parallel_loop(0, VREGS, 1, unroll=8)
                def _(i):
                    x = buf[i >> 5, pl.ds((i & 31) * LANES, LANES)]
                    off = (x * 4096.0).astype(jnp.int32) & 0xFFF
                    if region:
                        off = off | region
                    plsc.addupdate_scatter(hist, [off], ones)

                t_next = j * STEPS + s4 * NBUF + q + NBUF

                @pl.when(t_next < UNITS_PER_TILE * STEPS)
                def _():
                    s_next = s4 * NBUF + q + NBUF
                    j_next = j + s_next // STEPS
                    start_copy(q, row_of(j_next, lax.rem(s_next, STEPS)))
            return carry

        lax.fori_loop(0, STEPS // NBUF, quad_body, 0)

        @plsc.parallel_loop(0, 2 * NUM_BINS, 1, unroll=4)
        def _(g):
            row = hist[pl.ds(g * LANES, LANES)]
            s = jnp.sum(row)
            plsc.store_scatter(outbuf, [jnp.broadcast_to(g, (LANES,))],
                               jnp.broadcast_to(s, (LANES,)),
                               mask=lane == 0)

        u = u0 + j
        pltpu.sync_copy(outbuf,
                        out_hbm.at[pl.ds(u * 2 * NUM_BINS, 2 * NUM_BINS)])


@functools.partial(
    pl.kernel,
    mesh=plsc.VectorSubcoreMesh(core_axis_name="c", subcore_axis_name="s"),
    out_type=jax.ShapeDtypeStruct((96 * 2 * NUM_BINS,), jnp.float32),
    scratch_types=[
        pltpu.VMEM((ROWS, IMG_W), jnp.float32),
        pltpu.VMEM((ROWS, IMG_W), jnp.float32),
        pltpu.VMEM((ROWS, IMG_W), jnp.float32),
        pltpu.VMEM((ROWS, IMG_W), jnp.float32),
        pltpu.VMEM((2 * HSIZE,), jnp.float32),
        pltpu.VMEM((2 * NUM_BINS,), jnp.float32),
        pltpu.SemaphoreType.DMA,
        pltpu.SemaphoreType.DMA,
        pltpu.SemaphoreType.DMA,
        pltpu.SemaphoreType.DMA,
    ],
    compiler_params=pltpu.CompilerParams(needs_layout_passes=False),
)
def _hist_kernel(pred_hbm, target_hbm, out_hbm, buf0, buf1, buf2, buf3,
                 hist, outbuf, sem0, sem1, sem2, sem3):
    _hist_body(pred_hbm, target_hbm, out_hbm, buf0, buf1, buf2, buf3, hist,
               outbuf, sem0, sem1, sem2, sem3)


def kernel(pred, target):
    part = _hist_kernel(pred.reshape(N_CH * IMG_H, IMG_W),
                        target.reshape(N_CH * IMG_H, IMG_W))
    part = part.reshape(N_CH, 2, 2, NUM_BINS).sum(axis=1)
    p = part[:, 0, :]
    t = part[:, 1, :]
    p = p / (p.sum(axis=1, keepdims=True) + 1e-8)
    t = t / (t.sum(axis=1, keepdims=True) + 1e-8)
    return jnp.mean((p - t) ** 2)


# speedup vs baseline: 1.4196x; 1.4196x over previous
import functools

import jax
import jax.numpy as jnp
from jax import lax
from jax.experimental import pallas as pl
from jax.experimental.pallas import tpu as pltpu
from jax.experimental.pallas import tpu_sc as plsc

NUM_BINS = 256
IMG_B, IMG_C, IMG_H, IMG_W = 16, 3, 512, 512
N_CH = IMG_B * IMG_C
LANES = 16
UNITS_PER_TILE = 3
HALF_ROWS = IMG_H // 2
ROWS = 32
CHUNK = ROWS * IMG_W
N_CHUNKS = HALF_ROWS // ROWS
STEPS = 2 * N_CHUNKS
VREGS = CHUNK // LANES
HSIZE = NUM_BINS * LANES
NBUF = 4


def _hist_body(pred_hbm, target_hbm, out_hbm, buf0, buf1, buf2, buf3, hist,
               outbuf, sem0, sem1, sem2, sem3):
    cid = lax.axis_index("c")
    sid = lax.axis_index("s")
    wid = sid * 2 + cid
    lane = lax.iota(jnp.int32, LANES)
    lane_t = lane | (1 << 12)
    ones = jnp.ones((LANES,), jnp.float32)
    zeros = jnp.zeros((LANES,), jnp.float32)
    bufs = (buf0, buf1, buf2, buf3)
    sems = (sem0, sem1, sem2, sem3)
    u0 = wid * UNITS_PER_TILE

    def start_copy(q, row0):
        src = pred_hbm if q % 2 == 0 else target_hbm
        pltpu.async_copy(src.at[pl.ds(row0, ROWS), :], bufs[q], sems[q])

    def wait_copy(q):
        pltpu.make_async_copy(
            pred_hbm.at[pl.ds(0, ROWS), :], bufs[q], sems[q]).wait()

    def row_of(t_unit, t_step):
        return (u0 + t_unit) * HALF_ROWS + (t_step // 2) * ROWS

    for q in range(NBUF):
        start_copy(q, row_of(0, q))

    for j in range(UNITS_PER_TILE):

        @plsc.parallel_loop(0, 2 * NUM_BINS, 1, unroll=8)
        def _(k):
            hist[pl.ds(k * LANES, LANES)] = zeros

        def quad_body(s4, carry):
            for q in range(NBUF):
                wait_copy(q)
                buf = bufs[q]
                lane_c = lane if q % 2 == 0 else lane_t

                @plsc.parallel_loop(0, VREGS, 1, unroll=8)
                def _(i):
                    x = buf[i >> 5, pl.ds((i & 31) * LANES, LANES)]
                    off = (x * 4096.0).astype(jnp.int32)
                    off = (off & 0xFF0) | lane_c
                    plsc.addupdate_scatter(hist, [off], ones)

                t_next = j * STEPS + s4 * NBUF + q + NBUF

                @pl.when(t_next < UNITS_PER_TILE * STEPS)
                def _():
                    s_next = s4 * NBUF + q + NBUF
                    j_next = j + s_next // STEPS
                    start_copy(q, row_of(j_next, lax.rem(s_next, STEPS)))
            return carry

        lax.fori_loop(0, STEPS // NBUF, quad_body, 0)

        @plsc.parallel_loop(0, 2 * NUM_BINS, 1, unroll=4)
        def _(g):
            row = hist[pl.ds(g * LANES, LANES)]
            s = jnp.sum(row)
            plsc.store_scatter(outbuf, [jnp.broadcast_to(g, (LANES,))],
                               jnp.broadcast_to(s, (LANES,)),
                               mask=lane == 0)

        u = u0 + j
        pltpu.sync_copy(outbuf,
                        out_hbm.at[pl.ds(u * 2 * NUM_BINS, 2 * NUM_BINS)])


@functools.partial(
    pl.kernel,
    mesh=plsc.VectorSubcoreMesh(core_axis_name="c", subcore_axis_name="s"),
    out_type=jax.ShapeDtypeStruct((96 * 2 * NUM_BINS,), jnp.float32),
    scratch_types=[
        pltpu.VMEM((ROWS, IMG_W), jnp.float32),
        pltpu.VMEM((ROWS, IMG_W), jnp.float32),
        pltpu.VMEM((ROWS, IMG_W), jnp.float32),
        pltpu.VMEM((ROWS, IMG_W), jnp.float32),
        pltpu.VMEM((2 * HSIZE,), jnp.float32),
        pltpu.VMEM((2 * NUM_BINS,), jnp.float32),
        pltpu.SemaphoreType.DMA,
        pltpu.SemaphoreType.DMA,
        pltpu.SemaphoreType.DMA,
        pltpu.SemaphoreType.DMA,
    ],
    compiler_params=pltpu.CompilerParams(needs_layout_passes=False),
)
def _hist_kernel(pred_hbm, target_hbm, out_hbm, buf0, buf1, buf2, buf3,
                 hist, outbuf, sem0, sem1, sem2, sem3):
    _hist_body(pred_hbm, target_hbm, out_hbm, buf0, buf1, buf2, buf3, hist,
               outbuf, sem0, sem1, sem2, sem3)


def kernel(pred, target):
    part = _hist_kernel(pred.reshape(N_CH * IMG_H, IMG_W),
                        target.reshape(N_CH * IMG_H, IMG_W))
    part = part.reshape(N_CH, 2, 2, NUM_BINS).sum(axis=1)
    p = part[:, 0, :]
    t = part[:, 1, :]
    p = p / (p.sum(axis=1, keepdims=True) + 1e-8)
    t = t / (t.sum(axis=1, keepdims=True) + 1e-8)
    return jnp.mean((p - t) ** 2)
